# Initial kernel scaffold; baseline (speedup 1.0000x reference)
import jax
import jax.numpy as jnp
from jax.experimental import pallas as pl


def kernel(x, edge_index, W1, b1, W2, b2):
    raise NotImplementedError("write your pallas kernel here")


# speedup vs baseline: 9.3702x; 9.3702x over previous
import functools

import jax
import jax.numpy as jnp
from jax import lax
from jax.experimental import pallas as pl
from jax.experimental.pallas import tpu as pltpu
from jax.experimental.pallas import tpu_sc as plsc

N_NODES = 10000
D = 128
N_PAD = 10240
NC = 2
NS = 16
NW = NC * NS
NB = 80
B = 128
EPT = NB * B
E_PAD = NW * EPT

_mesh = plsc.VectorSubcoreMesh(core_axis_name="c", subcore_axis_name="s")

_SEG = N_PAD // NS


@functools.partial(
    pl.kernel,
    out_type=jax.ShapeDtypeStruct((NC, N_PAD), jnp.float32),
    mesh=_mesh,
    scratch_types=[
        pltpu.VMEM((NB, B), jnp.int32),
        pltpu.VMEM((B,), jnp.float32),
        pltpu.VMEM((_SEG,), jnp.float32),
        pltpu.VMEM_SHARED((N_PAD,), jnp.float32),
    ],
)
def _deg_kernel(dst_hbm, ones_hbm, zeros_hbm, degpart_hbm, dstv, onesv, bounce,
                deg_sh):
    c = lax.axis_index("c")
    s = lax.axis_index("s")
    tid = c * NS + s
    lo = s * _SEG
    pltpu.sync_copy(zeros_hbm.at[pl.ds(lo, _SEG)], deg_sh.at[pl.ds(lo, _SEG)])
    pltpu.sync_copy(ones_hbm, onesv)
    pltpu.sync_copy(dst_hbm.at[tid], dstv)
    plsc.subcore_barrier()

    def body(j, carry):
        pltpu.sync_copy(onesv, deg_sh.at[dstv.at[j]], add=True)
        return carry

    lax.fori_loop(0, NB, body, 0)
    plsc.subcore_barrier()
    pltpu.sync_copy(deg_sh.at[pl.ds(lo, _SEG)], bounce)
    pltpu.sync_copy(bounce, degpart_hbm.at[c, pl.ds(lo, _SEG)])


@functools.partial(
    pl.kernel,
    out_type=jax.ShapeDtypeStruct((NC, N_PAD, D), jnp.float32),
    mesh=_mesh,
    scratch_types=[
        pltpu.VMEM((NB, B), jnp.int32),
        pltpu.VMEM((NB, B), jnp.int32),
        pltpu.VMEM((B, D), jnp.float32),
        pltpu.VMEM_SHARED((N_PAD, D), jnp.float32),
        pltpu.SemaphoreType.DMA,
    ],
)
def _agg_kernel(hp_hbm, src_hbm, dst_hbm, zeros2_hbm, acc_hbm, srcv, dstv,
                rowbuf, acc_sh, gsem0):
    c = lax.axis_index("c")
    s = lax.axis_index("s")
    tid = c * NS + s
    lo = s * _SEG
    pltpu.sync_copy(zeros2_hbm.at[pl.ds(lo, _SEG)], acc_sh.at[pl.ds(lo, _SEG)])
    pltpu.sync_copy(src_hbm.at[tid], srcv)
    pltpu.sync_copy(dst_hbm.at[tid], dstv)
    plsc.subcore_barrier()

    def body(j, carry):
        pltpu.async_copy(hp_hbm.at[srcv.at[j]], rowbuf, gsem0).wait()
        pltpu.sync_copy(rowbuf, acc_sh.at[dstv.at[j]], add=True)
        return carry

    lax.fori_loop(0, NB, body, 0)
    plsc.subcore_barrier()

    def wbody(k, carry):
        pltpu.sync_copy(acc_sh.at[pl.ds(lo + k * B, B)], rowbuf)
        pltpu.sync_copy(rowbuf, acc_hbm.at[c, pl.ds(lo + k * B, B)])
        return carry

    lax.fori_loop(0, _SEG // B, wbody, 0)


def _tc1_body(degp, x_r, w_r, out_r):
    dis = lax.rsqrt(degp[0, :] + degp[1, :] + 1.0)
    h = jnp.dot(x_r[...], w_r[...], preferred_element_type=jnp.float32)
    out_r[...] = h * dis[:, None]


def _tc2_body(degp, a0, a1, hp1, b1r, w2r, out_r):
    dis = lax.rsqrt(degp[0, :] + degp[1, :] + 1.0)
    pre = (a0[...] + a1[...] + hp1[...]) * dis[:, None] + b1r[...]
    h2 = jnp.maximum(pre, 0.0)
    out_r[...] = jnp.dot(h2, w2r[...],
                         preferred_element_type=jnp.float32) * dis[:, None]


def _tc3_body(degp, a0, a1, hp2, b2r, out_r):
    dis = lax.rsqrt(degp[0, :] + degp[1, :] + 1.0)
    out_r[...] = (a0[...] + a1[...] + hp2[...]) * dis[:, None] + b2r[...]


_RB = 1024
_GRID = (N_PAD // _RB,)
_degp_spec = pl.BlockSpec((2, _RB), lambda i: (0, i))
_row_spec = pl.BlockSpec((_RB, D), lambda i: (i, 0))
_w_spec = pl.BlockSpec((D, D), lambda i: (0, 0))
_b_spec = pl.BlockSpec((1, D), lambda i: (0, 0))
_out_sds = jax.ShapeDtypeStruct((N_PAD, D), jnp.float32)

_tc1 = pl.pallas_call(
    _tc1_body, grid=_GRID,
    in_specs=[_degp_spec, _row_spec, _w_spec],
    out_specs=_row_spec, out_shape=_out_sds)

_tc2 = pl.pallas_call(
    _tc2_body, grid=_GRID,
    in_specs=[_degp_spec, _row_spec, _row_spec, _row_spec, _b_spec, _w_spec],
    out_specs=_row_spec, out_shape=_out_sds)

_tc3 = pl.pallas_call(
    _tc3_body, grid=_GRID,
    in_specs=[_degp_spec, _row_spec, _row_spec, _row_spec, _b_spec],
    out_specs=_row_spec, out_shape=_out_sds)


def kernel(x, edge_index, W1, b1, W2, b2):
    ei = edge_index.astype(jnp.int32)
    n_edges = ei.shape[1]
    pad = E_PAD - n_edges
    sink = jnp.full((pad,), N_NODES, dtype=jnp.int32)
    src_p = jnp.concatenate([ei[0], sink]).reshape(NW, NB, B)
    dst_p = jnp.concatenate([ei[1], sink]).reshape(NW, NB, B)
    x_pad = jnp.concatenate(
        [x, jnp.zeros((N_PAD - N_NODES, D), jnp.float32)], axis=0)
    ones_b = jnp.ones((B,), jnp.float32)
    zeros1 = jnp.zeros((N_PAD,), jnp.float32)
    zeros2 = jnp.zeros((N_PAD, D), jnp.float32)
    b1r = b1.reshape(1, D).astype(jnp.float32)
    b2r = b2.reshape(1, D).astype(jnp.float32)

    degpart = _deg_kernel(dst_p, ones_b, zeros1)
    hp1 = _tc1(degpart, x_pad, W1)
    acc1 = _agg_kernel(hp1, src_p, dst_p, zeros2)
    hp2 = _tc2(degpart, acc1[0], acc1[1], hp1, b1r, W2)
    acc2 = _agg_kernel(hp2, src_p, dst_p, zeros2)
    out = _tc3(degpart, acc2[0], acc2[1], hp2, b2r)
    return out[:N_NODES]


# speedup vs baseline: 10.2136x; 1.0900x over previous
import functools

import jax
import jax.numpy as jnp
from jax import lax
from jax.experimental import pallas as pl
from jax.experimental.pallas import tpu as pltpu
from jax.experimental.pallas import tpu_sc as plsc

N_NODES = 10000
D = 128
N_PAD = 10240
NC = 2
NS = 16
NW = NC * NS
NB = 80
B = 128
EPT = NB * B
E_PAD = NW * EPT

_mesh = plsc.VectorSubcoreMesh(core_axis_name="c", subcore_axis_name="s")

_SEG = N_PAD // NS


@functools.partial(
    pl.kernel,
    out_type=jax.ShapeDtypeStruct((NC, N_PAD), jnp.float32),
    mesh=_mesh,
    scratch_types=[
        pltpu.VMEM((NB, B), jnp.int32),
        pltpu.VMEM((B,), jnp.float32),
        pltpu.VMEM((_SEG,), jnp.float32),
        pltpu.VMEM_SHARED((N_PAD,), jnp.float32),
    ],
)
def _deg_kernel(dst_hbm, ones_hbm, zeros_hbm, degpart_hbm, dstv, onesv, bounce,
                deg_sh):
    c = lax.axis_index("c")
    s = lax.axis_index("s")
    tid = c * NS + s
    lo = s * _SEG
    pltpu.sync_copy(zeros_hbm.at[pl.ds(lo, _SEG)], deg_sh.at[pl.ds(lo, _SEG)])
    pltpu.sync_copy(ones_hbm, onesv)
    pltpu.sync_copy(dst_hbm.at[tid], dstv)
    plsc.subcore_barrier()

    def body(j, carry):
        pltpu.sync_copy(onesv, deg_sh.at[dstv.at[j]], add=True)
        return carry

    lax.fori_loop(0, NB, body, 0)
    plsc.subcore_barrier()
    pltpu.sync_copy(deg_sh.at[pl.ds(lo, _SEG)], bounce)
    pltpu.sync_copy(bounce, degpart_hbm.at[c, pl.ds(lo, _SEG)])


@functools.partial(
    pl.kernel,
    out_type=jax.ShapeDtypeStruct((NC, N_PAD, D), jnp.float32),
    mesh=_mesh,
    scratch_types=[
        pltpu.VMEM((NB // 2, B), jnp.int32),
        pltpu.VMEM((NB // 2, B), jnp.int32),
        pltpu.VMEM((2, B, D), jnp.float32),
        pltpu.VMEM_SHARED((N_PAD, D), jnp.float32),
        pltpu.SemaphoreType.DMA,
        pltpu.SemaphoreType.DMA,
    ],
)
def _agg_kernel(hp_hbm, src_hbm, dst_hbm, zeros2_hbm, acc_hbm, srcv, dstv,
                rowbuf, acc_sh, gsem0, gsem1):
    c = lax.axis_index("c")
    s = lax.axis_index("s")
    tid = c * NS + s
    lo = s * _SEG
    nbp = NB // 2
    pltpu.sync_copy(zeros2_hbm.at[pl.ds(lo, _SEG)], acc_sh.at[pl.ds(lo, _SEG)])
    plsc.subcore_barrier()

    for p in range(2):
        pltpu.sync_copy(src_hbm.at[tid, pl.ds(p * nbp, nbp)], srcv)
        pltpu.sync_copy(dst_hbm.at[tid, pl.ds(p * nbp, nbp)], dstv)
        pltpu.async_copy(hp_hbm.at[srcv.at[0]], rowbuf.at[0], gsem0)

        def body(g, carry):
            j0 = g * 2
            j1 = j0 + 1
            pltpu.async_copy(hp_hbm.at[srcv.at[j1]], rowbuf.at[1], gsem1)
            pltpu.make_async_copy(hp_hbm.at[srcv.at[j0]], rowbuf.at[0],
                                  gsem0).wait()
            pltpu.sync_copy(rowbuf.at[0], acc_sh.at[dstv.at[j0]], add=True)

            @pl.when(j1 + 1 < nbp)
            def _():
                pltpu.async_copy(hp_hbm.at[srcv.at[j1 + 1]], rowbuf.at[0],
                                 gsem0)

            pltpu.make_async_copy(hp_hbm.at[srcv.at[j1]], rowbuf.at[1],
                                  gsem1).wait()
            pltpu.sync_copy(rowbuf.at[1], acc_sh.at[dstv.at[j1]], add=True)
            return carry

        lax.fori_loop(0, nbp // 2, body, 0)

    plsc.subcore_barrier()

    def wbody(k, carry):
        pltpu.sync_copy(acc_sh.at[pl.ds(lo + k * B, B)], rowbuf.at[0])
        pltpu.sync_copy(rowbuf.at[0], acc_hbm.at[c, pl.ds(lo + k * B, B)])
        return carry

    lax.fori_loop(0, _SEG // B, wbody, 0)


def _tc1_body(degp, x_r, w_r, out_r):
    dis = lax.rsqrt(degp[0, :] + degp[1, :] + 1.0)
    h = jnp.dot(x_r[...], w_r[...], preferred_element_type=jnp.float32)
    out_r[...] = h * dis[:, None]


def _tc2_body(degp, a0, a1, hp1, b1r, w2r, out_r):
    dis = lax.rsqrt(degp[0, :] + degp[1, :] + 1.0)
    pre = (a0[...] + a1[...] + hp1[...]) * dis[:, None] + b1r[...]
    h2 = jnp.maximum(pre, 0.0)
    out_r[...] = jnp.dot(h2, w2r[...],
                         preferred_element_type=jnp.float32) * dis[:, None]


def _tc3_body(degp, a0, a1, hp2, b2r, out_r):
    dis = lax.rsqrt(degp[0, :] + degp[1, :] + 1.0)
    out_r[...] = (a0[...] + a1[...] + hp2[...]) * dis[:, None] + b2r[...]


_RB = 1024
_GRID = (N_PAD // _RB,)
_degp_spec = pl.BlockSpec((2, _RB), lambda i: (0, i))
_row_spec = pl.BlockSpec((_RB, D), lambda i: (i, 0))
_w_spec = pl.BlockSpec((D, D), lambda i: (0, 0))
_b_spec = pl.BlockSpec((1, D), lambda i: (0, 0))
_out_sds = jax.ShapeDtypeStruct((N_PAD, D), jnp.float32)

_tc1 = pl.pallas_call(
    _tc1_body, grid=_GRID,
    in_specs=[_degp_spec, _row_spec, _w_spec],
    out_specs=_row_spec, out_shape=_out_sds)

_tc2 = pl.pallas_call(
    _tc2_body, grid=_GRID,
    in_specs=[_degp_spec, _row_spec, _row_spec, _row_spec, _b_spec, _w_spec],
    out_specs=_row_spec, out_shape=_out_sds)

_tc3 = pl.pallas_call(
    _tc3_body, grid=_GRID,
    in_specs=[_degp_spec, _row_spec, _row_spec, _row_spec, _b_spec],
    out_specs=_row_spec, out_shape=_out_sds)


def kernel(x, edge_index, W1, b1, W2, b2):
    ei = edge_index.astype(jnp.int32)
    n_edges = ei.shape[1]
    pad = E_PAD - n_edges
    sink_src = jnp.full((pad,), N_NODES, dtype=jnp.int32)
    sink_dst = jnp.arange(pad, dtype=jnp.int32) % N_NODES
    src_p = jnp.concatenate([ei[0], sink_src]).reshape(NW, NB, B)
    dst_p = jnp.concatenate([ei[1], sink_dst]).reshape(NW, NB, B)
    dst_deg = jnp.concatenate([ei[1], sink_src]).reshape(NW, NB, B)
    x_pad = jnp.concatenate(
        [x, jnp.zeros((N_PAD - N_NODES, D), jnp.float32)], axis=0)
    ones_b = jnp.ones((B,), jnp.float32)
    zeros1 = jnp.zeros((N_PAD,), jnp.float32)
    zeros2 = jnp.zeros((N_PAD, D), jnp.float32)
    b1r = b1.reshape(1, D).astype(jnp.float32)
    b2r = b2.reshape(1, D).astype(jnp.float32)

    degpart = _deg_kernel(dst_deg, ones_b, zeros1)
    hp1 = _tc1(degpart, x_pad, W1)
    acc1 = _agg_kernel(hp1, src_p, dst_p, zeros2)
    hp2 = _tc2(degpart, acc1[0], acc1[1], hp1, b1r, W2)
    acc2 = _agg_kernel(hp2, src_p, dst_p, zeros2)
    out = _tc3(degpart, acc2[0], acc2[1], hp2, b2r)
    return out[:N_NODES]
